# Initial kernel scaffold; baseline (speedup 1.0000x reference)
import jax
import jax.numpy as jnp
from jax.experimental import pallas as pl


def kernel(tokens, embedding):
    raise NotImplementedError("write your pallas kernel here")


# speedup vs baseline: 1.0224x; 1.0224x over previous
import functools
import math

import jax
import jax.numpy as jnp
from jax import lax
from jax.experimental import pallas as pl
from jax.experimental.pallas import tpu as pltpu
from jax.experimental.pallas import tpu_sc as plsc

D_MODEL = 1024
LANES = 16
SCALE = math.sqrt(D_MODEL)


@functools.partial(jax.jit, static_argnums=(2, 3, 4))
def _sc_embed(tok, table, B, NC, NS):
    NW = NC * NS
    rows_per_w = B // NW
    C = 64
    nchunk = rows_per_w // C
    groups = D_MODEL // LANES

    mesh = plsc.VectorSubcoreMesh(core_axis_name="c", subcore_axis_name="s")

    @functools.partial(
        pl.kernel,
        out_type=jax.ShapeDtypeStruct((B, D_MODEL), jnp.float32),
        mesh=mesh,
        scratch_types=[
            pltpu.VMEM((nchunk, C), jnp.int32),
            pltpu.VMEM((C, D_MODEL), jnp.float32),
            pltpu.SemaphoreType.DMA,
        ],
    )
    def emb_kernel(tok_hbm, table_hbm, out_hbm, idx_v, buf, gsem):
        wid = lax.axis_index("s") * NC + lax.axis_index("c")
        base = wid * rows_per_w
        pltpu.sync_copy(tok_hbm.at[wid], idx_v)

        def chunk_body(j, carry):
            pltpu.async_copy(table_hbm.at[idx_v.at[j]], buf, gsem).wait()

            def row_body(r, c2):
                for q in range(groups):
                    sl = pl.ds(q * LANES, LANES)
                    buf[r, sl] = buf[r, sl] * SCALE
                return c2

            lax.fori_loop(0, C, row_body, 0, unroll=False)
            pltpu.sync_copy(buf, out_hbm.at[pl.ds(base + j * C, C)])
            return carry

        lax.fori_loop(0, nchunk, chunk_body, 0, unroll=False)

    tok_w = tok.reshape(NW, nchunk, C)
    return emb_kernel(tok_w, table)


def kernel(tokens, embedding):
    B = tokens.size
    try:
        info = plsc.get_sparse_core_info()
        NC, NS = info.num_cores, info.num_subcores
    except Exception:
        NC, NS = 2, 16
    tok = tokens.reshape(-1).astype(jnp.int32)
    out = _sc_embed(tok, embedding, B, NC, NS)
    return out.reshape(tokens.shape + (D_MODEL,))


# speedup vs baseline: 1.7242x; 1.6864x over previous
import functools
import math

import jax
import jax.numpy as jnp
from jax import lax
from jax.experimental import pallas as pl
from jax.experimental.pallas import tpu as pltpu
from jax.experimental.pallas import tpu_sc as plsc

D_MODEL = 1024
LANES = 16
SCALE = math.sqrt(D_MODEL)
NBUF = 4


@functools.partial(jax.jit, static_argnums=(2, 3, 4))
def _sc_embed(tok, table, B, NC, NS):
    NW = NC * NS
    rows_per_w = B // NW
    C = 16
    nchunk = rows_per_w // C
    groups = D_MODEL // LANES

    mesh = plsc.VectorSubcoreMesh(core_axis_name="c", subcore_axis_name="s")

    @functools.partial(
        pl.kernel,
        out_type=jax.ShapeDtypeStruct((B, D_MODEL), jnp.float32),
        mesh=mesh,
        scratch_types=[
            pltpu.VMEM((nchunk, C), jnp.int32),
            pltpu.VMEM((NBUF, C, D_MODEL), jnp.float32),
        ] + [pltpu.SemaphoreType.DMA] * (2 * NBUF),
    )
    def emb_kernel(tok_hbm, table_hbm, out_hbm, idx_v, bufs, *sems):
        gsem = sems[:NBUF]
        ssem = sems[NBUF:]
        wid = lax.axis_index("s") * NC + lax.axis_index("c")
        base = wid * rows_per_w
        pltpu.sync_copy(tok_hbm.at[wid], idx_v)

        def gather(j, b, sem):
            return pltpu.async_copy(
                table_hbm.at[idx_v.at[j]], bufs.at[b], sem)

        def store_desc(j, b, sem):
            return pltpu.make_async_copy(
                bufs.at[b], out_hbm.at[pl.ds(base + j * C, C)], sem)

        gather(0, 0, gsem[0])
        gather(1, 1, gsem[1])

        def step(j, b):
            b2 = (b + 2) % NBUF

            @pl.when(j >= 2)
            def _():
                store_desc(j - 2, b2, ssem[b2]).wait()

            @pl.when(j + 2 < nchunk)
            def _():
                gather(j + 2, b2, gsem[b2])

            pltpu.make_async_copy(
                table_hbm.at[idx_v.at[j]], bufs.at[b], gsem[b]).wait()

            def row_body(r, c2):
                for q in range(groups):
                    sl = pl.ds(q * LANES, LANES)
                    bufs[b, r, sl] = bufs[b, r, sl] * SCALE
                return c2

            lax.fori_loop(0, C, row_body, 0, unroll=False)
            store_desc(j, b, ssem[b]).start()

        def quad(j4, carry):
            for b in range(NBUF):
                step(j4 * NBUF + b, b)
            return carry

        lax.fori_loop(0, nchunk // NBUF, quad, 0, unroll=False)

        store_desc(nchunk - 2, (nchunk - 2) % NBUF,
                   ssem[(nchunk - 2) % NBUF]).wait()
        store_desc(nchunk - 1, (nchunk - 1) % NBUF,
                   ssem[(nchunk - 1) % NBUF]).wait()

    tok_w = tok.reshape(NW, nchunk, C)
    return emb_kernel(tok_w, table)


def kernel(tokens, embedding):
    B = tokens.size
    try:
        info = plsc.get_sparse_core_info()
        NC, NS = info.num_cores, info.num_subcores
    except Exception:
        NC, NS = 2, 16
    tok = tokens.reshape(-1).astype(jnp.int32)
    out = _sc_embed(tok, embedding, B, NC, NS)
    return out.reshape(tokens.shape + (D_MODEL,))
